# baseline (speedup 1.0000x reference)
import functools

import jax
import jax.numpy as jnp
from jax import lax
from jax.experimental import pallas as pl
from jax.experimental.pallas import tpu as pltpu
from jax.experimental.pallas import tpu_sc as plsc

N_NODES = 10000
N_EDGES = 320000
D_IN = 128
D_HID = 64
D_OUT = 128

EBLK = 128
NBLK = N_EDGES // EBLK
NC, NS = 2, 16
NW = NC * NS
BPT = NBLK // NW
TAIL = NBLK - NW * BPT
NBUF = 6
OUTER = BPT // NBUF
N_PAD = 10240
RPS = N_PAD // NS


def _sc_mesh():
    return plsc.VectorSubcoreMesh(
        core_axis_name="c", subcore_axis_name="s", num_cores=NC, num_subcores=NS
    )


def _sc_degree(ei3, ones_blk, zrows):
    @functools.partial(
        pl.kernel,
        out_type=jax.ShapeDtypeStruct((NC, N_PAD, 16), jnp.float32),
        mesh=_sc_mesh(),
        compiler_params=pltpu.CompilerParams(use_tc_tiling_on_sc=False),
        scratch_types=[
            pltpu.VMEM((BPT, EBLK), jnp.int32),
            pltpu.VMEM((EBLK, 16), jnp.float32),
            pltpu.VMEM((EBLK, 16), jnp.float32),
            pltpu.VMEM_SHARED((N_PAD, 16), jnp.float32),
            pltpu.SemaphoreType.DMA((NBUF,)),
        ],
    )
    def k(ei_hbm, ones_hbm, z_hbm, out_hbm, idx_v, ones_v, zst_v, acc, sem_s):
        c = lax.axis_index("c")
        s = lax.axis_index("s")
        w = c * NS + s
        base = w * BPT
        pltpu.sync_copy(ones_hbm, ones_v)
        pltpu.sync_copy(z_hbm, zst_v)
        for j in range(RPS // EBLK):
            pltpu.sync_copy(zst_v, acc.at[pl.ds(s * RPS + j * EBLK, EBLK)])
        pltpu.sync_copy(ei_hbm.at[1, pl.ds(base, BPT)], idx_v)
        plsc.subcore_barrier()

        def scat_wait(j):
            pltpu.make_async_copy(z_hbm, ones_v, sem_s.at[j]).wait()

        def outer(g, carry):
            for j in range(NBUF):
                @pl.when(g > 0)
                def _():
                    scat_wait(j)

                pltpu.async_copy(
                    ones_v, acc.at[idx_v.at[g * NBUF + j]], sem_s.at[j], add=True
                )
            return carry

        lax.fori_loop(0, OUTER, outer, None)
        for j in range(NBUF):
            scat_wait(j)

        @pl.when(w < TAIL)
        def _():
            pltpu.sync_copy(ei_hbm.at[1, pl.ds(NW * BPT + w, 1)], idx_v.at[pl.ds(0, 1)])
            pltpu.sync_copy(ones_v, acc.at[idx_v.at[0]], add=True)

        plsc.subcore_barrier()
        for j in range(RPS // EBLK):
            so = pl.ds(s * RPS + j * EBLK, EBLK)
            pltpu.sync_copy(acc.at[so], zst_v)
            pltpu.sync_copy(zst_v, out_hbm.at[c, so])

    return k(ei3, ones_blk, zrows)


def _sc_aggregate(ys, ei3, zrows):
    d = D_HID
    np_ = len(ys)

    @functools.partial(
        pl.kernel,
        out_type=jax.ShapeDtypeStruct((NC, N_PAD, 2 * d), jnp.float32),
        mesh=_sc_mesh(),
        compiler_params=pltpu.CompilerParams(use_tc_tiling_on_sc=False),
        scratch_types=[
            pltpu.VMEM((BPT, EBLK), jnp.int32),
            pltpu.VMEM((BPT, EBLK), jnp.int32),
            pltpu.VMEM((NBUF, EBLK, d), jnp.float32),
            pltpu.VMEM_SHARED((N_PAD, d), jnp.float32),
            pltpu.SemaphoreType.DMA((NBUF,)),
            pltpu.SemaphoreType.DMA((NBUF,)),
        ],
    )
    def k(*args):
        y_hbms = args[:np_]
        ei_hbm, z_hbm, out_hbm, sidx_v, didx_v, rows_v, acc, sem_g, sem_s = args[np_:]
        c = lax.axis_index("c")
        s = lax.axis_index("s")
        w = c * NS + s
        base = w * BPT
        pltpu.sync_copy(ei_hbm.at[0, pl.ds(base, BPT)], sidx_v)
        pltpu.sync_copy(ei_hbm.at[1, pl.ds(base, BPT)], didx_v)

        def gather_wait(y_hbm, j):
            pltpu.make_async_copy(
                y_hbm.at[pl.ds(0, EBLK)], rows_v.at[j], sem_g.at[j]
            ).wait()

        def scat_wait(y_hbm, j):
            pltpu.make_async_copy(
                y_hbm.at[pl.ds(0, EBLK)], rows_v.at[j], sem_s.at[j]
            ).wait()

        for p, y_hbm in enumerate(y_hbms):
            pltpu.sync_copy(z_hbm, rows_v.at[0])
            for j in range(RPS // EBLK):
                pltpu.sync_copy(rows_v.at[0], acc.at[pl.ds(s * RPS + j * EBLK, EBLK)])
            plsc.subcore_barrier()

            def outer(g, carry):
                for j in range(NBUF):
                    @pl.when(g > 0)
                    def _():
                        scat_wait(y_hbm, j)

                    pltpu.async_copy(
                        y_hbm.at[sidx_v.at[g * NBUF + j]], rows_v.at[j], sem_g.at[j]
                    )
                for j in range(NBUF):
                    gather_wait(y_hbm, j)
                    pltpu.async_copy(
                        rows_v.at[j], acc.at[didx_v.at[g * NBUF + j]],
                        sem_s.at[j], add=True,
                    )
                return carry

            lax.fori_loop(0, OUTER, outer, None)
            for j in range(NBUF):
                scat_wait(y_hbm, j)

            @pl.when(w < TAIL)
            def _():
                tb = pl.ds(NW * BPT + w, 1)
                pltpu.sync_copy(ei_hbm.at[0, tb], sidx_v.at[pl.ds(0, 1)])
                pltpu.sync_copy(ei_hbm.at[1, tb], didx_v.at[pl.ds(0, 1)])
                pltpu.async_copy(
                    y_hbm.at[sidx_v.at[0]], rows_v.at[0], sem_g.at[0]
                ).wait()
                pltpu.sync_copy(rows_v.at[0], acc.at[didx_v.at[0]], add=True)

            plsc.subcore_barrier()
            for j in range(RPS // EBLK):
                so = pl.ds(s * RPS + j * EBLK, EBLK)
                pltpu.sync_copy(acc.at[so], rows_v.at[0])
                pltpu.sync_copy(rows_v.at[0], out_hbm.at[c, so, pl.ds(d * p, d)])
            if p + 1 < np_ and TAIL > 0:
                pltpu.sync_copy(ei_hbm.at[0, pl.ds(base, 1)], sidx_v.at[pl.ds(0, 1)])
                pltpu.sync_copy(ei_hbm.at[1, pl.ds(base, 1)], didx_v.at[pl.ds(0, 1)])

    return k(*ys, ei3, zrows)


def _sc_aggregate_full(y, ei3, zrows):
    d = D_OUT
    RPS2 = N_NODES // NS
    NB2 = 3
    OUT2 = BPT // NB2

    @functools.partial(
        pl.kernel,
        out_type=jax.ShapeDtypeStruct((NC, N_NODES, d), jnp.float32),
        mesh=_sc_mesh(),
        compiler_params=pltpu.CompilerParams(use_tc_tiling_on_sc=False),
        scratch_types=[
            pltpu.VMEM((2, 2, NB2, EBLK), jnp.int32),
            pltpu.VMEM((NB2, EBLK, d), jnp.float32),
            pltpu.VMEM_SHARED((N_NODES, d), jnp.float32),
            pltpu.SemaphoreType.DMA((2,)),
            pltpu.SemaphoreType.DMA((NB2,)),
            pltpu.SemaphoreType.DMA((NB2,)),
        ],
    )
    def k(y_hbm, ei_hbm, z_hbm, out_hbm, ibuf, rows_v, acc, sem_i, sem_g, sem_s):
        c = lax.axis_index("c")
        s = lax.axis_index("s")
        w = c * NS + s
        base = w * BPT
        row0 = s * RPS2
        pltpu.sync_copy(z_hbm, rows_v.at[0])
        for j in range(4):
            pltpu.sync_copy(rows_v.at[0], acc.at[pl.ds(row0 + j * EBLK, EBLK)])
        pltpu.sync_copy(
            rows_v.at[0, pl.ds(0, RPS2 - 4 * EBLK)],
            acc.at[pl.ds(row0 + 4 * EBLK, RPS2 - 4 * EBLK)],
        )
        pltpu.async_copy(ei_hbm.at[:, pl.ds(base, NB2)], ibuf.at[0], sem_i.at[0])
        plsc.subcore_barrier()

        def idx_wait(b):
            pltpu.make_async_copy(
                ei_hbm.at[:, pl.ds(0, NB2)], ibuf.at[b], sem_i.at[b]
            ).wait()

        def gather_wait(j):
            pltpu.make_async_copy(
                y_hbm.at[pl.ds(0, EBLK)], rows_v.at[j], sem_g.at[j]
            ).wait()

        def scat_wait(j):
            pltpu.make_async_copy(
                y_hbm.at[pl.ds(0, EBLK)], rows_v.at[j], sem_s.at[j]
            ).wait()

        def outer(g2, carry):
            for half in range(2):
                g = 2 * g2 + half
                for j in range(NB2):
                    @pl.when(g > 0)
                    def _():
                        scat_wait(j)

                @pl.when(g + 1 < OUT2)
                def _():
                    pltpu.async_copy(
                        ei_hbm.at[:, pl.ds(base + (g + 1) * NB2, NB2)],
                        ibuf.at[1 - half],
                        sem_i.at[1 - half],
                    )

                idx_wait(half)
                for j in range(NB2):
                    pltpu.async_copy(
                        y_hbm.at[ibuf.at[half, 0, j]], rows_v.at[j], sem_g.at[j]
                    )
                for j in range(NB2):
                    gather_wait(j)
                    pltpu.async_copy(
                        rows_v.at[j], acc.at[ibuf.at[half, 1, j]], sem_s.at[j],
                        add=True,
                    )
            return carry

        lax.fori_loop(0, OUT2 // 2, outer, None)
        for j in range(NB2):
            scat_wait(j)

        @pl.when(w < TAIL)
        def _():
            tb = pl.ds(NW * BPT + w, 1)
            pltpu.sync_copy(ei_hbm.at[:, tb], ibuf.at[0, :, pl.ds(0, 1)])
            pltpu.async_copy(y_hbm.at[ibuf.at[0, 0, 0]], rows_v.at[0], sem_g.at[0]).wait()
            pltpu.sync_copy(rows_v.at[0], acc.at[ibuf.at[0, 1, 0]], add=True)

        plsc.subcore_barrier()
        for j in range(4):
            so = pl.ds(row0 + j * EBLK, EBLK)
            pltpu.sync_copy(acc.at[so], rows_v.at[0])
            pltpu.sync_copy(rows_v.at[0], out_hbm.at[c, so])
        so = pl.ds(row0 + 4 * EBLK, RPS2 - 4 * EBLK)
        pltpu.sync_copy(acc.at[so], rows_v.at[0, pl.ds(0, RPS2 - 4 * EBLK)])
        pltpu.sync_copy(rows_v.at[0, pl.ds(0, RPS2 - 4 * EBLK)], out_hbm.at[c, so])

    return k(y, ei3, zrows)


RB = 2000
GRID = N_NODES // RB


def _dis(p0, p1):
    return lax.rsqrt(1.0 + p0[:, 0:1] + p1[:, 0:1])


def _tc_xw1(x, W1):
    def body(x_ref, w_ref, o_ref):
        o_ref[...] = jnp.dot(x_ref[...], w_ref[...], preferred_element_type=jnp.float32)

    return pl.pallas_call(
        body,
        grid=(GRID,),
        in_specs=[
            pl.BlockSpec((RB, D_IN), lambda i: (i, 0)),
            pl.BlockSpec((D_IN, D_HID), lambda i: (0, 0)),
        ],
        out_specs=pl.BlockSpec((RB, D_HID), lambda i: (i, 0)),
        out_shape=jax.ShapeDtypeStruct((N_NODES, D_HID), jnp.float32),
    )(x, W1)


def _tc_scale(xw1, degp):
    def body(x_ref, dp_ref, o_ref):
        o_ref[...] = _dis(dp_ref[0], dp_ref[1]) * x_ref[...]

    return pl.pallas_call(
        body,
        grid=(GRID,),
        in_specs=[
            pl.BlockSpec((RB, D_HID), lambda i: (i, 0)),
            pl.BlockSpec((NC, RB, 16), lambda i: (0, i, 0)),
        ],
        out_specs=pl.BlockSpec((RB, D_HID), lambda i: (i, 0)),
        out_shape=jax.ShapeDtypeStruct((N_NODES, D_HID), jnp.float32),
    )(xw1, degp)


def _tc_layer2(q, y1, b1, W2, degp):
    def body(q_ref, y1_ref, b1_ref, w_ref, dp_ref, o_ref):
        dis = _dis(dp_ref[0], dp_ref[1])
        qs = q_ref[0] + q_ref[1]
        h = jnp.maximum(dis * (qs[:, :D_HID] + y1_ref[...]) + b1_ref[...], 0.0)
        o_ref[...] = dis * jnp.dot(h, w_ref[...], preferred_element_type=jnp.float32)

    return pl.pallas_call(
        body,
        grid=(GRID,),
        in_specs=[
            pl.BlockSpec((NC, RB, 2 * D_HID), lambda i: (0, i, 0)),
            pl.BlockSpec((RB, D_HID), lambda i: (i, 0)),
            pl.BlockSpec((1, D_HID), lambda i: (0, 0)),
            pl.BlockSpec((D_HID, D_OUT), lambda i: (0, 0)),
            pl.BlockSpec((NC, RB, 16), lambda i: (0, i, 0)),
        ],
        out_specs=pl.BlockSpec((RB, D_OUT), lambda i: (i, 0)),
        out_shape=jax.ShapeDtypeStruct((N_NODES, D_OUT), jnp.float32),
    )(q, y1, b1, W2, degp)


def _tc_out(r, y2, b2, degp):
    def body(r_ref, y2_ref, b2_ref, dp_ref, o_ref):
        dis = _dis(dp_ref[0], dp_ref[1])
        o_ref[...] = dis * (r_ref[0] + r_ref[1] + y2_ref[...]) + b2_ref[...]

    return pl.pallas_call(
        body,
        grid=(GRID,),
        in_specs=[
            pl.BlockSpec((NC, RB, D_OUT), lambda i: (0, i, 0)),
            pl.BlockSpec((RB, D_OUT), lambda i: (i, 0)),
            pl.BlockSpec((1, D_OUT), lambda i: (0, 0)),
            pl.BlockSpec((NC, RB, 16), lambda i: (0, i, 0)),
        ],
        out_specs=pl.BlockSpec((RB, D_OUT), lambda i: (i, 0)),
        out_shape=jax.ShapeDtypeStruct((N_NODES, D_OUT), jnp.float32),
    )(r, y2, b2, degp)


def kernel(x, edge_index, W1, b1, W2, b2):
    ei3 = edge_index.astype(jnp.int32).reshape(2, NBLK, EBLK)
    ones_blk = jnp.ones((EBLK, 16), jnp.float32)
    z16 = jnp.zeros((EBLK, 16), jnp.float32)
    z64 = jnp.zeros((EBLK, D_HID), jnp.float32)
    z128 = jnp.zeros((EBLK, D_OUT), jnp.float32)

    xw1 = _tc_xw1(x, W1)
    degp = _sc_degree(ei3, ones_blk, z16)
    y1 = _tc_scale(xw1, degp)
    q = _sc_aggregate([y1], ei3, z64)
    y2 = _tc_layer2(q, y1, b1.reshape(1, D_HID), W2, degp)
    r = _sc_aggregate_full(y2, ei3, z128)
    return _tc_out(r, y2, b2.reshape(1, D_OUT), degp)


# speedup vs baseline: 1.0760x; 1.0760x over previous
import functools

import jax
import jax.numpy as jnp
from jax import lax
from jax.experimental import pallas as pl
from jax.experimental.pallas import tpu as pltpu
from jax.experimental.pallas import tpu_sc as plsc

N_NODES = 10000
N_EDGES = 320000
D_IN = 128
D_HID = 64
D_OUT = 128

EBLK = 128
NBLK = N_EDGES // EBLK
NC, NS = 2, 16
NW = NC * NS
BPT = NBLK // NW
TAIL = NBLK - NW * BPT
NBUF = 6
OUTER = BPT // NBUF
N_PAD = 10240
RPS = N_PAD // NS


def _sc_mesh():
    return plsc.VectorSubcoreMesh(
        core_axis_name="c", subcore_axis_name="s", num_cores=NC, num_subcores=NS
    )


def _sc_degree(ei3, ones_blk, zrows):
    @functools.partial(
        pl.kernel,
        out_type=jax.ShapeDtypeStruct((NC, N_PAD, 16), jnp.float32),
        mesh=_sc_mesh(),
        compiler_params=pltpu.CompilerParams(use_tc_tiling_on_sc=False),
        scratch_types=[
            pltpu.VMEM((BPT, EBLK), jnp.int32),
            pltpu.VMEM((EBLK, 16), jnp.float32),
            pltpu.VMEM((EBLK, 16), jnp.float32),
            pltpu.VMEM_SHARED((N_PAD, 16), jnp.float32),
            pltpu.SemaphoreType.DMA((NBUF,)),
        ],
    )
    def k(ei_hbm, ones_hbm, z_hbm, out_hbm, idx_v, ones_v, zst_v, acc, sem_s):
        c = lax.axis_index("c")
        s = lax.axis_index("s")
        w = c * NS + s
        base = w * BPT
        pltpu.sync_copy(ones_hbm, ones_v)
        pltpu.sync_copy(z_hbm, zst_v)
        for j in range(RPS // EBLK):
            pltpu.sync_copy(zst_v, acc.at[pl.ds(s * RPS + j * EBLK, EBLK)])
        pltpu.sync_copy(ei_hbm.at[1, pl.ds(base, BPT)], idx_v)
        plsc.subcore_barrier()

        def scat_wait(j):
            pltpu.make_async_copy(z_hbm, ones_v, sem_s.at[j]).wait()

        def outer(g, carry):
            for j in range(NBUF):
                @pl.when(g > 0)
                def _():
                    scat_wait(j)

                pltpu.async_copy(
                    ones_v, acc.at[idx_v.at[g * NBUF + j]], sem_s.at[j], add=True
                )
            return carry

        lax.fori_loop(0, OUTER, outer, None)
        for j in range(NBUF):
            scat_wait(j)

        @pl.when(w < TAIL)
        def _():
            pltpu.sync_copy(ei_hbm.at[1, pl.ds(NW * BPT + w, 1)], idx_v.at[pl.ds(0, 1)])
            pltpu.sync_copy(ones_v, acc.at[idx_v.at[0]], add=True)

        plsc.subcore_barrier()
        for j in range(RPS // EBLK):
            so = pl.ds(s * RPS + j * EBLK, EBLK)
            pltpu.sync_copy(acc.at[so], zst_v)
            pltpu.sync_copy(zst_v, out_hbm.at[c, so])

    return k(ei3, ones_blk, zrows)


def _sc_aggregate(ys, ei3, zrows):
    d = D_HID
    np_ = len(ys)

    @functools.partial(
        pl.kernel,
        out_type=jax.ShapeDtypeStruct((NC, N_PAD, 2 * d), jnp.float32),
        mesh=_sc_mesh(),
        compiler_params=pltpu.CompilerParams(use_tc_tiling_on_sc=False),
        scratch_types=[
            pltpu.VMEM((BPT, EBLK), jnp.int32),
            pltpu.VMEM((BPT, EBLK), jnp.int32),
            pltpu.VMEM((NBUF, EBLK, d), jnp.float32),
            pltpu.VMEM_SHARED((N_PAD, d), jnp.float32),
            pltpu.SemaphoreType.DMA((NBUF,)),
            pltpu.SemaphoreType.DMA((NBUF,)),
        ],
    )
    def k(*args):
        y_hbms = args[:np_]
        ei_hbm, z_hbm, out_hbm, sidx_v, didx_v, rows_v, acc, sem_g, sem_s = args[np_:]
        c = lax.axis_index("c")
        s = lax.axis_index("s")
        w = c * NS + s
        base = w * BPT
        pltpu.sync_copy(ei_hbm.at[0, pl.ds(base, BPT)], sidx_v)
        pltpu.sync_copy(ei_hbm.at[1, pl.ds(base, BPT)], didx_v)

        def gather_wait(y_hbm, j):
            pltpu.make_async_copy(
                y_hbm.at[pl.ds(0, EBLK)], rows_v.at[j], sem_g.at[j]
            ).wait()

        def scat_wait(y_hbm, j):
            pltpu.make_async_copy(
                y_hbm.at[pl.ds(0, EBLK)], rows_v.at[j], sem_s.at[j]
            ).wait()

        for p, y_hbm in enumerate(y_hbms):
            pltpu.sync_copy(z_hbm, rows_v.at[0])
            for j in range(RPS // EBLK):
                pltpu.sync_copy(rows_v.at[0], acc.at[pl.ds(s * RPS + j * EBLK, EBLK)])
            plsc.subcore_barrier()

            def outer(g, carry):
                for j in range(NBUF):
                    @pl.when(g > 0)
                    def _():
                        scat_wait(y_hbm, j)

                    pltpu.async_copy(
                        y_hbm.at[sidx_v.at[g * NBUF + j]], rows_v.at[j], sem_g.at[j]
                    )
                for j in range(NBUF):
                    gather_wait(y_hbm, j)
                    pltpu.async_copy(
                        rows_v.at[j], acc.at[didx_v.at[g * NBUF + j]],
                        sem_s.at[j], add=True,
                    )
                return carry

            lax.fori_loop(0, OUTER, outer, None)
            for j in range(NBUF):
                scat_wait(y_hbm, j)

            @pl.when(w < TAIL)
            def _():
                tb = pl.ds(NW * BPT + w, 1)
                pltpu.sync_copy(ei_hbm.at[0, tb], sidx_v.at[pl.ds(0, 1)])
                pltpu.sync_copy(ei_hbm.at[1, tb], didx_v.at[pl.ds(0, 1)])
                pltpu.async_copy(
                    y_hbm.at[sidx_v.at[0]], rows_v.at[0], sem_g.at[0]
                ).wait()
                pltpu.sync_copy(rows_v.at[0], acc.at[didx_v.at[0]], add=True)

            plsc.subcore_barrier()
            for j in range(RPS // EBLK):
                so = pl.ds(s * RPS + j * EBLK, EBLK)
                pltpu.sync_copy(acc.at[so], rows_v.at[0])
                pltpu.sync_copy(rows_v.at[0], out_hbm.at[c, so, pl.ds(d * p, d)])
            if p + 1 < np_ and TAIL > 0:
                pltpu.sync_copy(ei_hbm.at[0, pl.ds(base, 1)], sidx_v.at[pl.ds(0, 1)])
                pltpu.sync_copy(ei_hbm.at[1, pl.ds(base, 1)], didx_v.at[pl.ds(0, 1)])

    return k(*ys, ei3, zrows)


RB = 2000
GRID = N_NODES // RB


def _dis(p0, p1):
    return lax.rsqrt(1.0 + p0[:, 0:1] + p1[:, 0:1])


def _tc_xw1(x, W1):
    def body(x_ref, w_ref, o_ref):
        o_ref[...] = jnp.dot(x_ref[...], w_ref[...], preferred_element_type=jnp.float32)

    return pl.pallas_call(
        body,
        grid=(GRID,),
        in_specs=[
            pl.BlockSpec((RB, D_IN), lambda i: (i, 0)),
            pl.BlockSpec((D_IN, D_HID), lambda i: (0, 0)),
        ],
        out_specs=pl.BlockSpec((RB, D_HID), lambda i: (i, 0)),
        out_shape=jax.ShapeDtypeStruct((N_NODES, D_HID), jnp.float32),
    )(x, W1)


def _tc_scale(xw1, degp):
    def body(x_ref, dp_ref, o_ref):
        o_ref[...] = _dis(dp_ref[0], dp_ref[1]) * x_ref[...]

    return pl.pallas_call(
        body,
        grid=(GRID,),
        in_specs=[
            pl.BlockSpec((RB, D_HID), lambda i: (i, 0)),
            pl.BlockSpec((NC, RB, 16), lambda i: (0, i, 0)),
        ],
        out_specs=pl.BlockSpec((RB, D_HID), lambda i: (i, 0)),
        out_shape=jax.ShapeDtypeStruct((N_NODES, D_HID), jnp.float32),
    )(xw1, degp)


def _tc_layer2(q, y1, b1, W2, degp):
    def body(q_ref, y1_ref, b1_ref, w_ref, dp_ref, lo_ref, hi_ref):
        dis = _dis(dp_ref[0], dp_ref[1])
        qs = q_ref[0] + q_ref[1]
        h = jnp.maximum(dis * (qs[:, :D_HID] + y1_ref[...]) + b1_ref[...], 0.0)
        y2 = dis * jnp.dot(h, w_ref[...], preferred_element_type=jnp.float32)
        lo_ref[...] = y2[:, :D_HID]
        hi_ref[...] = y2[:, D_HID:]

    return pl.pallas_call(
        body,
        grid=(GRID,),
        in_specs=[
            pl.BlockSpec((NC, RB, 2 * D_HID), lambda i: (0, i, 0)),
            pl.BlockSpec((RB, D_HID), lambda i: (i, 0)),
            pl.BlockSpec((1, D_HID), lambda i: (0, 0)),
            pl.BlockSpec((D_HID, D_OUT), lambda i: (0, 0)),
            pl.BlockSpec((NC, RB, 16), lambda i: (0, i, 0)),
        ],
        out_specs=[
            pl.BlockSpec((RB, D_HID), lambda i: (i, 0)),
            pl.BlockSpec((RB, D_HID), lambda i: (i, 0)),
        ],
        out_shape=[
            jax.ShapeDtypeStruct((N_NODES, D_HID), jnp.float32),
            jax.ShapeDtypeStruct((N_NODES, D_HID), jnp.float32),
        ],
    )(q, y1, b1, W2, degp)


def _tc_out(r, y2lo, y2hi, b2, degp):
    def body(r_ref, lo_ref, hi_ref, b2_ref, dp_ref, o_ref):
        dis = _dis(dp_ref[0], dp_ref[1])
        rsum = r_ref[0] + r_ref[1]
        olo = dis * (rsum[:, :D_HID] + lo_ref[...])
        ohi = dis * (rsum[:, D_HID:] + hi_ref[...])
        o_ref[...] = jnp.concatenate([olo, ohi], axis=1) + b2_ref[...]

    return pl.pallas_call(
        body,
        grid=(GRID,),
        in_specs=[
            pl.BlockSpec((NC, RB, D_OUT), lambda i: (0, i, 0)),
            pl.BlockSpec((RB, D_HID), lambda i: (i, 0)),
            pl.BlockSpec((RB, D_HID), lambda i: (i, 0)),
            pl.BlockSpec((1, D_OUT), lambda i: (0, 0)),
            pl.BlockSpec((NC, RB, 16), lambda i: (0, i, 0)),
        ],
        out_specs=pl.BlockSpec((RB, D_OUT), lambda i: (i, 0)),
        out_shape=jax.ShapeDtypeStruct((N_NODES, D_OUT), jnp.float32),
    )(r, y2lo, y2hi, b2, degp)


def kernel(x, edge_index, W1, b1, W2, b2):
    ei3 = edge_index.astype(jnp.int32).reshape(2, NBLK, EBLK)
    ones_blk = jnp.ones((EBLK, 16), jnp.float32)
    z16 = jnp.zeros((EBLK, 16), jnp.float32)
    z64 = jnp.zeros((EBLK, D_HID), jnp.float32)

    xw1 = _tc_xw1(x, W1)
    degp = _sc_degree(ei3, ones_blk, z16)
    y1 = _tc_scale(xw1, degp)
    q = _sc_aggregate([y1], ei3, z64)
    y2lo, y2hi = _tc_layer2(q, y1, b1.reshape(1, D_HID), W2, degp)
    r = _sc_aggregate([y2lo, y2hi], ei3, z64)
    return _tc_out(r, y2lo, y2hi, b2.reshape(1, D_OUT), degp)


# speedup vs baseline: 1.1833x; 1.0997x over previous
import functools

import jax
import jax.numpy as jnp
from jax import lax
from jax.experimental import pallas as pl
from jax.experimental.pallas import tpu as pltpu
from jax.experimental.pallas import tpu_sc as plsc

N_NODES = 10000
N_EDGES = 320000
D_IN = 128
D_HID = 64
D_OUT = 128

EBLK = 128
NBLK = N_EDGES // EBLK
NC, NS = 2, 16
NW = NC * NS
BPT = NBLK // NW
TAIL = NBLK - NW * BPT
NBUF = 6
OUTER = BPT // NBUF
N_PAD = 10240
RPS = N_PAD // NS


def _sc_mesh():
    return plsc.VectorSubcoreMesh(
        core_axis_name="c", subcore_axis_name="s", num_cores=NC, num_subcores=NS
    )


def _sc_degree(ei3, ones_blk, zrows):
    @functools.partial(
        pl.kernel,
        out_type=jax.ShapeDtypeStruct((NC, N_PAD, 16), jnp.float32),
        mesh=_sc_mesh(),
        compiler_params=pltpu.CompilerParams(use_tc_tiling_on_sc=False),
        scratch_types=[
            pltpu.VMEM((BPT, EBLK), jnp.int32),
            pltpu.VMEM((EBLK, 16), jnp.float32),
            pltpu.VMEM((EBLK, 16), jnp.float32),
            pltpu.VMEM_SHARED((N_PAD, 16), jnp.float32),
            pltpu.SemaphoreType.DMA((NBUF,)),
        ],
    )
    def k(ei_hbm, ones_hbm, z_hbm, out_hbm, idx_v, ones_v, zst_v, acc, sem_s):
        c = lax.axis_index("c")
        s = lax.axis_index("s")
        w = c * NS + s
        base = w * BPT
        pltpu.sync_copy(ones_hbm, ones_v)
        pltpu.sync_copy(z_hbm, zst_v)
        for j in range(RPS // EBLK):
            pltpu.sync_copy(zst_v, acc.at[pl.ds(s * RPS + j * EBLK, EBLK)])
        pltpu.sync_copy(ei_hbm.at[1, pl.ds(base, BPT)], idx_v)
        plsc.subcore_barrier()

        def scat_wait(j):
            pltpu.make_async_copy(z_hbm, ones_v, sem_s.at[j]).wait()

        def outer(g, carry):
            for j in range(NBUF):
                @pl.when(g > 0)
                def _():
                    scat_wait(j)

                pltpu.async_copy(
                    ones_v, acc.at[idx_v.at[g * NBUF + j]], sem_s.at[j], add=True
                )
            return carry

        lax.fori_loop(0, OUTER, outer, None)
        for j in range(NBUF):
            scat_wait(j)

        @pl.when(w < TAIL)
        def _():
            pltpu.sync_copy(ei_hbm.at[1, pl.ds(NW * BPT + w, 1)], idx_v.at[pl.ds(0, 1)])
            pltpu.sync_copy(ones_v, acc.at[idx_v.at[0]], add=True)

        plsc.subcore_barrier()
        for j in range(RPS // EBLK):
            so = pl.ds(s * RPS + j * EBLK, EBLK)
            pltpu.sync_copy(acc.at[so], zst_v)
            pltpu.sync_copy(zst_v, out_hbm.at[c, so])

    return k(ei3, ones_blk, zrows)


def _sc_aggregate(ys, ei3, zrows):
    d = D_HID
    np_ = len(ys)

    @functools.partial(
        pl.kernel,
        out_type=jax.ShapeDtypeStruct((NC, N_PAD, 2 * d), jnp.bfloat16),
        mesh=_sc_mesh(),
        compiler_params=pltpu.CompilerParams(use_tc_tiling_on_sc=False),
        scratch_types=[
            pltpu.VMEM((BPT, EBLK), jnp.int32),
            pltpu.VMEM((BPT, EBLK), jnp.int32),
            pltpu.VMEM((NBUF, EBLK, d), jnp.bfloat16),
            pltpu.VMEM_SHARED((N_PAD, d), jnp.bfloat16),
            pltpu.SemaphoreType.DMA((NBUF,)),
            pltpu.SemaphoreType.DMA((NBUF,)),
        ],
    )
    def k(*args):
        y_hbms = args[:np_]
        ei_hbm, z_hbm, out_hbm, sidx_v, didx_v, rows_v, acc, sem_g, sem_s = args[np_:]
        c = lax.axis_index("c")
        s = lax.axis_index("s")
        w = c * NS + s
        base = w * BPT
        pltpu.sync_copy(ei_hbm.at[0, pl.ds(base, BPT)], sidx_v)
        pltpu.sync_copy(ei_hbm.at[1, pl.ds(base, BPT)], didx_v)

        def gather_wait(y_hbm, j):
            pltpu.make_async_copy(
                y_hbm.at[pl.ds(0, EBLK)], rows_v.at[j], sem_g.at[j]
            ).wait()

        def scat_wait(y_hbm, j):
            pltpu.make_async_copy(
                y_hbm.at[pl.ds(0, EBLK)], rows_v.at[j], sem_s.at[j]
            ).wait()

        for p, y_hbm in enumerate(y_hbms):
            pltpu.sync_copy(z_hbm, rows_v.at[0])
            for j in range(RPS // EBLK):
                pltpu.sync_copy(rows_v.at[0], acc.at[pl.ds(s * RPS + j * EBLK, EBLK)])
            plsc.subcore_barrier()

            def outer(g, carry):
                for j in range(NBUF):
                    @pl.when(g > 0)
                    def _():
                        scat_wait(y_hbm, j)

                    pltpu.async_copy(
                        y_hbm.at[sidx_v.at[g * NBUF + j]], rows_v.at[j], sem_g.at[j]
                    )
                for j in range(NBUF):
                    gather_wait(y_hbm, j)
                    pltpu.async_copy(
                        rows_v.at[j], acc.at[didx_v.at[g * NBUF + j]],
                        sem_s.at[j], add=True,
                    )
                return carry

            lax.fori_loop(0, OUTER, outer, None)
            for j in range(NBUF):
                scat_wait(y_hbm, j)

            @pl.when(w < TAIL)
            def _():
                tb = pl.ds(NW * BPT + w, 1)
                pltpu.sync_copy(ei_hbm.at[0, tb], sidx_v.at[pl.ds(0, 1)])
                pltpu.sync_copy(ei_hbm.at[1, tb], didx_v.at[pl.ds(0, 1)])
                pltpu.async_copy(
                    y_hbm.at[sidx_v.at[0]], rows_v.at[0], sem_g.at[0]
                ).wait()
                pltpu.sync_copy(rows_v.at[0], acc.at[didx_v.at[0]], add=True)

            plsc.subcore_barrier()
            for j in range(RPS // EBLK):
                so = pl.ds(s * RPS + j * EBLK, EBLK)
                pltpu.sync_copy(acc.at[so], rows_v.at[0])
                pltpu.sync_copy(rows_v.at[0], out_hbm.at[c, so, pl.ds(d * p, d)])
            if p + 1 < np_ and TAIL > 0:
                pltpu.sync_copy(ei_hbm.at[0, pl.ds(base, 1)], sidx_v.at[pl.ds(0, 1)])
                pltpu.sync_copy(ei_hbm.at[1, pl.ds(base, 1)], didx_v.at[pl.ds(0, 1)])

    return k(*ys, ei3, zrows)


RB = 2000
GRID = N_NODES // RB


def _dis(p0, p1):
    return lax.rsqrt(1.0 + p0[:, 0:1] + p1[:, 0:1])


def _tc_xw1(x, W1):
    def body(x_ref, w_ref, o_ref):
        o_ref[...] = jnp.dot(x_ref[...], w_ref[...], preferred_element_type=jnp.float32)

    return pl.pallas_call(
        body,
        grid=(GRID,),
        in_specs=[
            pl.BlockSpec((RB, D_IN), lambda i: (i, 0)),
            pl.BlockSpec((D_IN, D_HID), lambda i: (0, 0)),
        ],
        out_specs=pl.BlockSpec((RB, D_HID), lambda i: (i, 0)),
        out_shape=jax.ShapeDtypeStruct((N_NODES, D_HID), jnp.float32),
    )(x, W1)


def _tc_scale(xw1, degp):
    def body(x_ref, dp_ref, o_ref):
        o_ref[...] = (_dis(dp_ref[0], dp_ref[1]) * x_ref[...]).astype(jnp.bfloat16)

    return pl.pallas_call(
        body,
        grid=(GRID,),
        in_specs=[
            pl.BlockSpec((RB, D_HID), lambda i: (i, 0)),
            pl.BlockSpec((NC, RB, 16), lambda i: (0, i, 0)),
        ],
        out_specs=pl.BlockSpec((RB, D_HID), lambda i: (i, 0)),
        out_shape=jax.ShapeDtypeStruct((N_NODES, D_HID), jnp.bfloat16),
    )(xw1, degp)


def _tc_layer2(q, y1, b1, W2, degp):
    def body(q_ref, y1_ref, b1_ref, w_ref, dp_ref, lo_ref, hi_ref):
        dis = _dis(dp_ref[0], dp_ref[1])
        qs = q_ref[0].astype(jnp.float32) + q_ref[1].astype(jnp.float32)
        h = jnp.maximum(
            dis * (qs[:, :D_HID] + y1_ref[...].astype(jnp.float32)) + b1_ref[...], 0.0
        )
        y2 = dis * jnp.dot(h, w_ref[...], preferred_element_type=jnp.float32)
        lo_ref[...] = y2[:, :D_HID].astype(jnp.bfloat16)
        hi_ref[...] = y2[:, D_HID:].astype(jnp.bfloat16)

    return pl.pallas_call(
        body,
        grid=(GRID,),
        in_specs=[
            pl.BlockSpec((NC, RB, 2 * D_HID), lambda i: (0, i, 0)),
            pl.BlockSpec((RB, D_HID), lambda i: (i, 0)),
            pl.BlockSpec((1, D_HID), lambda i: (0, 0)),
            pl.BlockSpec((D_HID, D_OUT), lambda i: (0, 0)),
            pl.BlockSpec((NC, RB, 16), lambda i: (0, i, 0)),
        ],
        out_specs=[
            pl.BlockSpec((RB, D_HID), lambda i: (i, 0)),
            pl.BlockSpec((RB, D_HID), lambda i: (i, 0)),
        ],
        out_shape=[
            jax.ShapeDtypeStruct((N_NODES, D_HID), jnp.bfloat16),
            jax.ShapeDtypeStruct((N_NODES, D_HID), jnp.bfloat16),
        ],
    )(q, y1, b1, W2, degp)


def _tc_out(r, y2lo, y2hi, b2, degp):
    def body(r_ref, lo_ref, hi_ref, b2_ref, dp_ref, o_ref):
        dis = _dis(dp_ref[0], dp_ref[1])
        rsum = r_ref[0].astype(jnp.float32) + r_ref[1].astype(jnp.float32)
        olo = dis * (rsum[:, :D_HID] + lo_ref[...].astype(jnp.float32))
        ohi = dis * (rsum[:, D_HID:] + hi_ref[...].astype(jnp.float32))
        o_ref[...] = jnp.concatenate([olo, ohi], axis=1) + b2_ref[...]

    return pl.pallas_call(
        body,
        grid=(GRID,),
        in_specs=[
            pl.BlockSpec((NC, RB, D_OUT), lambda i: (0, i, 0)),
            pl.BlockSpec((RB, D_HID), lambda i: (i, 0)),
            pl.BlockSpec((RB, D_HID), lambda i: (i, 0)),
            pl.BlockSpec((1, D_OUT), lambda i: (0, 0)),
            pl.BlockSpec((NC, RB, 16), lambda i: (0, i, 0)),
        ],
        out_specs=pl.BlockSpec((RB, D_OUT), lambda i: (i, 0)),
        out_shape=jax.ShapeDtypeStruct((N_NODES, D_OUT), jnp.float32),
    )(r, y2lo, y2hi, b2, degp)


def kernel(x, edge_index, W1, b1, W2, b2):
    ei3 = edge_index.astype(jnp.int32).reshape(2, NBLK, EBLK)
    ones_blk = jnp.ones((EBLK, 16), jnp.float32)
    z16 = jnp.zeros((EBLK, 16), jnp.float32)
    z64 = jnp.zeros((EBLK, D_HID), jnp.bfloat16)

    xw1 = _tc_xw1(x, W1)
    degp = _sc_degree(ei3, ones_blk, z16)
    y1 = _tc_scale(xw1, degp)
    q = _sc_aggregate([y1], ei3, z64)
    y2lo, y2hi = _tc_layer2(q, y1, b1.reshape(1, D_HID), W2, degp)
    r = _sc_aggregate([y2lo, y2hi], ei3, z64)
    return _tc_out(r, y2lo, y2hi, b2.reshape(1, D_OUT), degp)
